# Initial kernel scaffold; baseline (speedup 1.0000x reference)
import jax
import jax.numpy as jnp
from jax.experimental import pallas as pl


def kernel(x, x_bi, sadj, W1, b1, W2, b2, dec_W1, dec_b1, dec_W2, dec_b2, zW, zb, bn_gamma, bn_beta, piW, pib, dispW, dispb, meanW, meanb):
    raise NotImplementedError("write your pallas kernel here")


# speedup vs baseline: 1.8003x; 1.8003x over previous
import functools

import jax
import jax.numpy as jnp
from jax.experimental import pallas as pl
from jax.experimental.pallas import tpu as pltpu

N = 10000
ROWS = 400
EPS = 1e-5


def _supports_body(x_ref, xbi_ref, w1_ref, out_ref):
    w1 = w1_ref[...]
    a = jnp.dot(x_ref[...], w1, preferred_element_type=jnp.float32)
    b = jnp.dot(xbi_ref[...], w1, preferred_element_type=jnp.float32)
    out_ref[...] = jnp.concatenate([a, b], axis=1)


def _pass1_body(sadj_ref, s1_ref, b1c_ref, w2c_ref, out_ref):
    a = sadj_ref[...].astype(jnp.bfloat16)
    s = s1_ref[...].astype(jnp.bfloat16)
    t = jnp.dot(a, s, preferred_element_type=jnp.float32)
    h = jax.nn.relu(t + b1c_ref[...])
    out_ref[...] = jnp.dot(h, w2c_ref[...], preferred_element_type=jnp.float32)


def _pass2_body(sadj_ref, h2_ref, b2c_ref, dw1p_ref, db1_ref, dw2_ref,
                db2_ref, zwp_ref, zb_ref,
                emb_ref, embbi_ref, de_ref, z_ref):
    a = sadj_ref[...].astype(jnp.bfloat16)
    h = h2_ref[...].astype(jnp.bfloat16)
    e = jnp.dot(a, h, preferred_element_type=jnp.float32) + b2c_ref[...]
    emb_ref[...] = e[:, :32]
    embbi_ref[...] = e[:, 32:]
    d1 = jax.nn.relu(
        jnp.dot(e, dw1p_ref[...], preferred_element_type=jnp.float32)
        + db1_ref[...])
    de_ref[...] = (jnp.dot(d1, dw2_ref[...], preferred_element_type=jnp.float32)
                   + db2_ref[...])
    z_ref[...] = (jnp.dot(e, zwp_ref[...], preferred_element_type=jnp.float32)
                  + zb_ref[...])


def _heads_body(z_ref, g_ref, bta_ref, piw_ref, pib_ref, dw_ref, db_ref,
                mw_ref, mb_ref, pi_ref, disp_ref, mean_ref):
    z = z_ref[...]
    mu = jnp.mean(z, axis=0, keepdims=True)
    var = jnp.mean((z - mu) ** 2, axis=0, keepdims=True)
    zn = (z - mu) / jnp.sqrt(var + EPS) * g_ref[...] + bta_ref[...]
    zr = jax.nn.relu(zn)
    pi_ref[...] = jax.nn.sigmoid(
        jnp.dot(zr, piw_ref[...], preferred_element_type=jnp.float32)
        + pib_ref[...])
    t = (jnp.dot(zr, dw_ref[...], preferred_element_type=jnp.float32)
         + db_ref[...])
    sp = jnp.maximum(t, 0.0) + jnp.log1p(jnp.exp(-jnp.abs(t)))
    disp_ref[...] = jnp.clip(sp, 0.0001, 10000.0)
    m = (jnp.dot(zr, mw_ref[...], preferred_element_type=jnp.float32)
         + mb_ref[...])
    mean_ref[...] = jnp.clip(jnp.exp(m), 1e-05, 1000000.0)


@functools.partial(jax.jit, static_argnums=())
def kernel(x, x_bi, sadj, W1, b1, W2, b2, dec_W1, dec_b1, dec_W2, dec_b2,
           zW, zb, bn_gamma, bn_beta, piW, pib, dispW, dispb, meanW, meanb):
    f32 = jnp.float32

    b1c = jnp.concatenate([b1, b1]).reshape(1, 128)
    w2c = jnp.zeros((128, 64), f32).at[:64, :32].set(W2).at[64:, 32:].set(W2)
    b2c = jnp.concatenate([b2, b2]).reshape(1, 64)
    dw1p = jnp.zeros((64, 64), f32).at[:32, :].set(dec_W1)
    zwp = jnp.zeros((64, 64), f32).at[32:, :].set(zW)

    s1cat = pl.pallas_call(
        _supports_body,
        out_shape=jax.ShapeDtypeStruct((N, 128), f32),
    )(x, x_bi, W1)

    h2cat = pl.pallas_call(
        _pass1_body,
        grid=(N // ROWS,),
        in_specs=[
            pl.BlockSpec((ROWS, N), lambda i: (i, 0)),
            pl.BlockSpec((N, 128), lambda i: (0, 0)),
            pl.BlockSpec((1, 128), lambda i: (0, 0)),
            pl.BlockSpec((128, 64), lambda i: (0, 0)),
        ],
        out_specs=pl.BlockSpec((ROWS, 64), lambda i: (i, 0)),
        out_shape=jax.ShapeDtypeStruct((N, 64), f32),
        compiler_params=pltpu.CompilerParams(
            dimension_semantics=("arbitrary",)),
    )(sadj, s1cat, b1c, w2c)

    emb, emb_bi, de_emb, z = pl.pallas_call(
        _pass2_body,
        grid=(N // ROWS,),
        in_specs=[
            pl.BlockSpec((ROWS, N), lambda i: (i, 0)),
            pl.BlockSpec((N, 64), lambda i: (0, 0)),
            pl.BlockSpec((1, 64), lambda i: (0, 0)),
            pl.BlockSpec((64, 64), lambda i: (0, 0)),
            pl.BlockSpec((1, 64), lambda i: (0, 0)),
            pl.BlockSpec((64, 128), lambda i: (0, 0)),
            pl.BlockSpec((1, 128), lambda i: (0, 0)),
            pl.BlockSpec((64, 64), lambda i: (0, 0)),
            pl.BlockSpec((1, 64), lambda i: (0, 0)),
        ],
        out_specs=[
            pl.BlockSpec((ROWS, 32), lambda i: (i, 0)),
            pl.BlockSpec((ROWS, 32), lambda i: (i, 0)),
            pl.BlockSpec((ROWS, 128), lambda i: (i, 0)),
            pl.BlockSpec((ROWS, 64), lambda i: (i, 0)),
        ],
        out_shape=[
            jax.ShapeDtypeStruct((N, 32), f32),
            jax.ShapeDtypeStruct((N, 32), f32),
            jax.ShapeDtypeStruct((N, 128), f32),
            jax.ShapeDtypeStruct((N, 64), f32),
        ],
        compiler_params=pltpu.CompilerParams(
            dimension_semantics=("arbitrary",)),
    )(sadj, h2cat, b2c, dw1p, dec_b1.reshape(1, 64), dec_W2,
      dec_b2.reshape(1, 128), zwp, zb.reshape(1, 64))

    pi, disp, mean = pl.pallas_call(
        _heads_body,
        out_shape=[
            jax.ShapeDtypeStruct((N, 128), f32),
            jax.ShapeDtypeStruct((N, 128), f32),
            jax.ShapeDtypeStruct((N, 128), f32),
        ],
    )(z, bn_gamma.reshape(1, 64), bn_beta.reshape(1, 64), piW,
      pib.reshape(1, 128), dispW, dispb.reshape(1, 128), meanW,
      meanb.reshape(1, 128))

    return (emb, emb_bi, de_emb, pi, disp, mean)
